# baseline (speedup 1.0000x reference)
import functools

import jax
import jax.numpy as jnp
from jax import lax
from jax.experimental import pallas as pl
from jax.experimental.pallas import tpu as pltpu
from jax.experimental.pallas import tpu_sc as plsc

NC, NS = 2, 16
NW = NC * NS
CHUNK = 64
D = 128
DA = 144
BLK = 640
K0, K1 = 236, 80
KH = max(K0, K1) // 2


def _sc_pass_body(stripe, tbl, gidx, sidx, zeros, out, gv, sv, vals0, vals1,
                  acc, sem):
    cid = lax.axis_index("c")
    sid = lax.axis_index("s")
    wid = cid * NS + sid
    n_acc = acc.shape[0]
    kr = jnp.where(cid == 0, K0 // 2, K1 // 2)
    kq = jnp.where(cid == 0, K0 // 4, K1 // 4)
    pltpu.sync_copy(zeros.at[pl.ds(sid * stripe, stripe)],
                    acc.at[pl.ds(sid * stripe, stripe)])
    plsc.subcore_barrier()

    for r in range(2):
        pltpu.sync_copy(gidx.at[wid].at[pl.ds(r * kr, KH)], gv)
        pltpu.sync_copy(sidx.at[wid].at[pl.ds(r * kr, KH)], sv)
        pltpu.async_copy(tbl.at[gv.at[0]], vals0, sem)

        def step(i, carry):
            j = 2 * i
            pltpu.make_async_copy(tbl.at[gv.at[j]], vals0, sem).wait()
            pltpu.async_copy(tbl.at[gv.at[j + 1]], vals1, sem)
            pltpu.sync_copy(vals0, acc.at[sv.at[j]], add=True)
            pltpu.make_async_copy(tbl.at[gv.at[j + 1]], vals1, sem).wait()

            @pl.when(j + 2 < kr)
            def _():
                pltpu.async_copy(tbl.at[gv.at[j + 2]], vals0, sem)

            pltpu.sync_copy(vals1, acc.at[sv.at[j + 1]], add=True)
            return carry

        lax.fori_loop(0, kq, step, 0)

    plsc.subcore_barrier()
    pltpu.sync_copy(acc.at[pl.ds(sid * stripe, stripe)],
                    out.at[pl.ds(cid * n_acc + sid * stripe, stripe)])


def _make_sc_pass(n_acc):
    stripe = n_acc // NS
    mesh = plsc.VectorSubcoreMesh(core_axis_name="c", subcore_axis_name="s",
                                  num_cores=NC, num_subcores=NS)
    return pl.kernel(
        functools.partial(_sc_pass_body, stripe),
        out_type=jax.ShapeDtypeStruct((NC * n_acc, DA), jnp.float32),
        mesh=mesh,
        scratch_types=[
            pltpu.VMEM((KH, CHUNK), jnp.int32),
            pltpu.VMEM((KH, CHUNK), jnp.int32),
            pltpu.VMEM((CHUNK, DA), jnp.float32),
            pltpu.VMEM((CHUNK, DA), jnp.float32),
            pltpu.VMEM_SHARED((n_acc, DA), jnp.float32),
            pltpu.SemaphoreType.DMA,
        ],
        compiler_params=pltpu.CompilerParams(use_tc_tiling_on_sc=False),
    )


def _ones_col(rows):
    return (lax.broadcasted_iota(jnp.int32, (rows, DA - D), 1) == 0).astype(jnp.float32)


def _k1_body(x_ref, w_ref, o_ref):
    m = jnp.dot(x_ref[...], w_ref[...], preferred_element_type=jnp.float32)
    o_ref[...] = jnp.concatenate([m, _ones_col(m.shape[0])], axis=1)


def _k2_body(acc_ref, w_ref, o_ref):
    p = acc_ref[0] + acc_ref[1]
    ef = p[:, :D] / jnp.maximum(p[:, D:D + 1], 1.0)
    ep = jnp.dot(ef, w_ref[...], preferred_element_type=jnp.float32)
    o_ref[...] = jnp.concatenate([ep, _ones_col(ep.shape[0])], axis=1)


def _k3_body(xa_ref, acc_ref, wu_ref, b_ref, o_ref):
    p = acc_ref[0] + acc_ref[1]
    nagg = p[:, :D] / jnp.maximum(p[:, D:D + 1], 1.0)
    h = (jnp.dot(xa_ref[:, :D], wu_ref[:D], preferred_element_type=jnp.float32)
         + jnp.dot(nagg, wu_ref[D:], preferred_element_type=jnp.float32)
         + b_ref[...])
    o_ref[...] = jnp.maximum(h, 0.0)


def _split_edges(idx, n, e):
    e0 = NS * K0 * CHUNK
    e1 = NS * K1 * CHUNK
    flat = jnp.concatenate([idx, jnp.full((e0 + e1 - e,), n, jnp.int32)])
    part0 = flat[:e0].reshape(NS, K0, CHUNK)
    part0 = jnp.pad(part0, ((0, 0), (0, max(K0, K1) - K0), (0, 0)),
                    constant_values=n)
    part1 = flat[e0:].reshape(NS, K1, CHUNK)
    part1 = jnp.pad(part1, ((0, 0), (0, max(K0, K1) - K1), (0, 0)),
                    constant_values=n)
    return jnp.concatenate([part0, part1], axis=0)


def kernel(x, edge_index, W_v, W_e, W_upd, b_upd):
    n = x.shape[0]
    e = edge_index.shape[1]
    n_pad = ((n + 1 + BLK - 1) // BLK) * BLK
    n_acc = ((n + 1 + NS - 1) // NS) * NS

    row_p = _split_edges(edge_index[0], n, e)
    col_p = _split_edges(edge_index[1], n, e)

    x_pad = jnp.zeros((n_pad, D), jnp.float32).at[:n].set(x)
    zeros = jnp.zeros((n_acc, DA), jnp.float32)

    grid = n_pad // BLK
    full = lambda shape: pl.BlockSpec(shape, lambda i: (0,) * len(shape))

    x_self_aug = pl.pallas_call(
        _k1_body,
        grid=(grid,),
        in_specs=[pl.BlockSpec((BLK, D), lambda i: (i, 0)), full((D, D))],
        out_specs=pl.BlockSpec((BLK, DA), lambda i: (i, 0)),
        out_shape=jax.ShapeDtypeStruct((n_pad, DA), jnp.float32),
    )(x_pad, W_v)

    sc_pass = _make_sc_pass(n_acc)
    acc_a = sc_pass(x_self_aug, row_p, col_p, zeros).reshape(NC, n_acc, DA)

    e_proj_aug = pl.pallas_call(
        _k2_body,
        grid=(grid,),
        in_specs=[pl.BlockSpec((NC, BLK, DA), lambda i: (0, i, 0)), full((D, D))],
        out_specs=pl.BlockSpec((BLK, DA), lambda i: (i, 0)),
        out_shape=jax.ShapeDtypeStruct((n_pad, DA), jnp.float32),
    )(acc_a, W_e)

    acc_b = sc_pass(e_proj_aug, col_p, row_p, zeros).reshape(NC, n_acc, DA)

    out = pl.pallas_call(
        _k3_body,
        grid=(grid,),
        in_specs=[
            pl.BlockSpec((BLK, DA), lambda i: (i, 0)),
            pl.BlockSpec((NC, BLK, DA), lambda i: (0, i, 0)),
            full((2 * D, D)),
            full((1, D)),
        ],
        out_specs=pl.BlockSpec((BLK, D), lambda i: (i, 0)),
        out_shape=jax.ShapeDtypeStruct((n_pad, D), jnp.float32),
    )(x_self_aug, acc_b, W_upd, b_upd.reshape(1, D))

    return out[:n]


# speedup vs baseline: 1.0004x; 1.0004x over previous
import functools

import jax
import jax.numpy as jnp
from jax import lax
from jax.experimental import pallas as pl
from jax.experimental.pallas import tpu as pltpu
from jax.experimental.pallas import tpu_sc as plsc

NC, NS = 2, 16
NW = NC * NS
CHUNK = 64
D = 128
DA = 144
BLK = 640
K0, K1 = 224, 92
KH = max(K0, K1) // 2


def _sc_pass_body(stripe, tbl, gidx, sidx, zeros, out, gv, sv, vals0, vals1,
                  acc, sem):
    cid = lax.axis_index("c")
    sid = lax.axis_index("s")
    wid = cid * NS + sid
    n_acc = acc.shape[0]
    kr = jnp.where(cid == 0, K0 // 2, K1 // 2)
    kq = jnp.where(cid == 0, K0 // 4, K1 // 4)
    pltpu.sync_copy(zeros.at[pl.ds(sid * stripe, stripe)],
                    acc.at[pl.ds(sid * stripe, stripe)])
    plsc.subcore_barrier()

    for r in range(2):
        pltpu.sync_copy(gidx.at[wid].at[pl.ds(r * kr, KH)], gv)
        pltpu.sync_copy(sidx.at[wid].at[pl.ds(r * kr, KH)], sv)
        pltpu.async_copy(tbl.at[gv.at[0]], vals0, sem)

        def step(i, carry):
            j = 2 * i
            pltpu.make_async_copy(tbl.at[gv.at[j]], vals0, sem).wait()
            pltpu.async_copy(tbl.at[gv.at[j + 1]], vals1, sem)
            pltpu.sync_copy(vals0, acc.at[sv.at[j]], add=True)
            pltpu.make_async_copy(tbl.at[gv.at[j + 1]], vals1, sem).wait()

            @pl.when(j + 2 < kr)
            def _():
                pltpu.async_copy(tbl.at[gv.at[j + 2]], vals0, sem)

            pltpu.sync_copy(vals1, acc.at[sv.at[j + 1]], add=True)
            return carry

        lax.fori_loop(0, kq, step, 0)

    plsc.subcore_barrier()
    pltpu.sync_copy(acc.at[pl.ds(sid * stripe, stripe)],
                    out.at[pl.ds(cid * n_acc + sid * stripe, stripe)])


def _make_sc_pass(n_acc):
    stripe = n_acc // NS
    mesh = plsc.VectorSubcoreMesh(core_axis_name="c", subcore_axis_name="s",
                                  num_cores=NC, num_subcores=NS)
    return pl.kernel(
        functools.partial(_sc_pass_body, stripe),
        out_type=jax.ShapeDtypeStruct((NC * n_acc, DA), jnp.float32),
        mesh=mesh,
        scratch_types=[
            pltpu.VMEM((KH, CHUNK), jnp.int32),
            pltpu.VMEM((KH, CHUNK), jnp.int32),
            pltpu.VMEM((CHUNK, DA), jnp.float32),
            pltpu.VMEM((CHUNK, DA), jnp.float32),
            pltpu.VMEM_SHARED((n_acc, DA), jnp.float32),
            pltpu.SemaphoreType.DMA,
        ],
        compiler_params=pltpu.CompilerParams(use_tc_tiling_on_sc=False),
    )


def _ones_col(rows):
    return (lax.broadcasted_iota(jnp.int32, (rows, DA - D), 1) == 0).astype(jnp.float32)


def _k1_body(x_ref, w_ref, o_ref):
    m = jnp.dot(x_ref[...], w_ref[...], preferred_element_type=jnp.float32)
    o_ref[...] = jnp.concatenate([m, _ones_col(m.shape[0])], axis=1)


def _k2_body(acc_ref, w_ref, o_ref):
    p = acc_ref[0] + acc_ref[1]
    ef = p[:, :D] / jnp.maximum(p[:, D:D + 1], 1.0)
    ep = jnp.dot(ef, w_ref[...], preferred_element_type=jnp.float32)
    o_ref[...] = jnp.concatenate([ep, _ones_col(ep.shape[0])], axis=1)


def _k3_body(xa_ref, acc_ref, wu_ref, b_ref, o_ref):
    p = acc_ref[0] + acc_ref[1]
    nagg = p[:, :D] / jnp.maximum(p[:, D:D + 1], 1.0)
    h = (jnp.dot(xa_ref[:, :D], wu_ref[:D], preferred_element_type=jnp.float32)
         + jnp.dot(nagg, wu_ref[D:], preferred_element_type=jnp.float32)
         + b_ref[...])
    o_ref[...] = jnp.maximum(h, 0.0)


def _split_edges(idx, n, e):
    e0 = NS * K0 * CHUNK
    e1 = NS * K1 * CHUNK
    flat = jnp.concatenate([idx, jnp.full((e0 + e1 - e,), n, jnp.int32)])
    part0 = flat[:e0].reshape(NS, K0, CHUNK)
    part0 = jnp.pad(part0, ((0, 0), (0, max(K0, K1) - K0), (0, 0)),
                    constant_values=n)
    part1 = flat[e0:].reshape(NS, K1, CHUNK)
    part1 = jnp.pad(part1, ((0, 0), (0, max(K0, K1) - K1), (0, 0)),
                    constant_values=n)
    return jnp.concatenate([part0, part1], axis=0)


def kernel(x, edge_index, W_v, W_e, W_upd, b_upd):
    n = x.shape[0]
    e = edge_index.shape[1]
    n_pad = ((n + 1 + BLK - 1) // BLK) * BLK
    n_acc = ((n + 1 + NS - 1) // NS) * NS

    row_p = _split_edges(edge_index[0], n, e)
    col_p = _split_edges(edge_index[1], n, e)

    x_pad = jnp.zeros((n_pad, D), jnp.float32).at[:n].set(x)
    zeros = jnp.zeros((n_acc, DA), jnp.float32)

    grid = n_pad // BLK
    full = lambda shape: pl.BlockSpec(shape, lambda i: (0,) * len(shape))

    x_self_aug = pl.pallas_call(
        _k1_body,
        grid=(grid,),
        in_specs=[pl.BlockSpec((BLK, D), lambda i: (i, 0)), full((D, D))],
        out_specs=pl.BlockSpec((BLK, DA), lambda i: (i, 0)),
        out_shape=jax.ShapeDtypeStruct((n_pad, DA), jnp.float32),
    )(x_pad, W_v)

    sc_pass = _make_sc_pass(n_acc)
    acc_a = sc_pass(x_self_aug, row_p, col_p, zeros).reshape(NC, n_acc, DA)

    e_proj_aug = pl.pallas_call(
        _k2_body,
        grid=(grid,),
        in_specs=[pl.BlockSpec((NC, BLK, DA), lambda i: (0, i, 0)), full((D, D))],
        out_specs=pl.BlockSpec((BLK, DA), lambda i: (i, 0)),
        out_shape=jax.ShapeDtypeStruct((n_pad, DA), jnp.float32),
    )(acc_a, W_e)

    acc_b = sc_pass(e_proj_aug, col_p, row_p, zeros).reshape(NC, n_acc, DA)

    out = pl.pallas_call(
        _k3_body,
        grid=(grid,),
        in_specs=[
            pl.BlockSpec((BLK, DA), lambda i: (i, 0)),
            pl.BlockSpec((NC, BLK, DA), lambda i: (0, i, 0)),
            full((2 * D, D)),
            full((1, D)),
        ],
        out_specs=pl.BlockSpec((BLK, D), lambda i: (i, 0)),
        out_shape=jax.ShapeDtypeStruct((n_pad, D), jnp.float32),
    )(x_self_aug, acc_b, W_upd, b_upd.reshape(1, D))

    return out[:n]


# speedup vs baseline: 1.0710x; 1.0706x over previous
import functools

import jax
import jax.numpy as jnp
from jax import lax
from jax.experimental import pallas as pl
from jax.experimental.pallas import tpu as pltpu
from jax.experimental.pallas import tpu_sc as plsc

NC, NS = 2, 16
NW = NC * NS
CHUNK = 64
D = 128
DA = 144
BLK = 640
K0, K1 = 212, 104
KH = max(K0, K1) // 2


def _sc_pass_body(stripe, tbl, gidx, sidx, zeros, out, gv, sv, vals0, vals1,
                  acc, sem):
    cid = lax.axis_index("c")
    sid = lax.axis_index("s")
    wid = cid * NS + sid
    n_acc = acc.shape[0]
    kr = jnp.where(cid == 0, K0 // 2, K1 // 2)
    kq = jnp.where(cid == 0, K0 // 4, K1 // 4)
    pltpu.sync_copy(zeros.at[pl.ds(sid * stripe, stripe)],
                    acc.at[pl.ds(sid * stripe, stripe)])
    plsc.subcore_barrier()

    for r in range(2):
        pltpu.sync_copy(gidx.at[wid].at[pl.ds(r * kr, KH)], gv)
        pltpu.sync_copy(sidx.at[wid].at[pl.ds(r * kr, KH)], sv)
        pltpu.async_copy(tbl.at[gv.at[0]], vals0, sem)

        def step(i, carry):
            j = 2 * i
            pltpu.make_async_copy(tbl.at[gv.at[j]], vals0, sem).wait()
            pltpu.async_copy(tbl.at[gv.at[j + 1]], vals1, sem)
            pltpu.sync_copy(vals0, acc.at[sv.at[j]], add=True)
            pltpu.make_async_copy(tbl.at[gv.at[j + 1]], vals1, sem).wait()

            @pl.when(j + 2 < kr)
            def _():
                pltpu.async_copy(tbl.at[gv.at[j + 2]], vals0, sem)

            pltpu.sync_copy(vals1, acc.at[sv.at[j + 1]], add=True)
            return carry

        lax.fori_loop(0, kq, step, 0)

    plsc.subcore_barrier()
    pltpu.sync_copy(acc.at[pl.ds(sid * stripe, stripe)],
                    out.at[pl.ds(cid * n_acc + sid * stripe, stripe)])


def _make_sc_pass(n_acc):
    stripe = n_acc // NS
    mesh = plsc.VectorSubcoreMesh(core_axis_name="c", subcore_axis_name="s",
                                  num_cores=NC, num_subcores=NS)
    return pl.kernel(
        functools.partial(_sc_pass_body, stripe),
        out_type=jax.ShapeDtypeStruct((NC * n_acc, DA), jnp.float32),
        mesh=mesh,
        scratch_types=[
            pltpu.VMEM((KH, CHUNK), jnp.int32),
            pltpu.VMEM((KH, CHUNK), jnp.int32),
            pltpu.VMEM((CHUNK, DA), jnp.float32),
            pltpu.VMEM((CHUNK, DA), jnp.float32),
            pltpu.VMEM_SHARED((n_acc, DA), jnp.float32),
            pltpu.SemaphoreType.DMA,
        ],
        compiler_params=pltpu.CompilerParams(use_tc_tiling_on_sc=False),
    )


def _ones_col(rows):
    return (lax.broadcasted_iota(jnp.int32, (rows, DA - D), 1) == 0).astype(jnp.float32)


def _k1_body(x_ref, w_ref, o_ref):
    m = jnp.dot(x_ref[...], w_ref[...], preferred_element_type=jnp.float32)
    o_ref[...] = jnp.concatenate([m, _ones_col(m.shape[0])], axis=1)


def _k2_body(acc_ref, w_ref, o_ref):
    p = acc_ref[0] + acc_ref[1]
    ef = p[:, :D] / jnp.maximum(p[:, D:D + 1], 1.0)
    ep = jnp.dot(ef, w_ref[...], preferred_element_type=jnp.float32)
    o_ref[...] = jnp.concatenate([ep, _ones_col(ep.shape[0])], axis=1)


def _k3_body(xa_ref, acc_ref, wu_ref, b_ref, o_ref):
    p = acc_ref[0] + acc_ref[1]
    nagg = p[:, :D] / jnp.maximum(p[:, D:D + 1], 1.0)
    h = (jnp.dot(xa_ref[:, :D], wu_ref[:D], preferred_element_type=jnp.float32)
         + jnp.dot(nagg, wu_ref[D:], preferred_element_type=jnp.float32)
         + b_ref[...])
    o_ref[...] = jnp.maximum(h, 0.0)


def _split_edges(idx, n, e):
    e0 = NS * K0 * CHUNK
    e1 = NS * K1 * CHUNK
    flat = jnp.concatenate([idx, jnp.full((e0 + e1 - e,), n, jnp.int32)])
    part0 = flat[:e0].reshape(NS, K0, CHUNK)
    part0 = jnp.pad(part0, ((0, 0), (0, max(K0, K1) - K0), (0, 0)),
                    constant_values=n)
    part1 = flat[e0:].reshape(NS, K1, CHUNK)
    part1 = jnp.pad(part1, ((0, 0), (0, max(K0, K1) - K1), (0, 0)),
                    constant_values=n)
    return jnp.concatenate([part0, part1], axis=0)


def kernel(x, edge_index, W_v, W_e, W_upd, b_upd):
    n = x.shape[0]
    e = edge_index.shape[1]
    n_pad = ((n + 1 + BLK - 1) // BLK) * BLK
    n_acc = ((n + 1 + NS - 1) // NS) * NS

    row_p = _split_edges(edge_index[0], n, e)
    col_p = _split_edges(edge_index[1], n, e)

    x_pad = jnp.zeros((n_pad, D), jnp.float32).at[:n].set(x)
    zeros = jnp.zeros((n_acc, DA), jnp.float32)

    grid = n_pad // BLK
    full = lambda shape: pl.BlockSpec(shape, lambda i: (0,) * len(shape))

    x_self_aug = pl.pallas_call(
        _k1_body,
        grid=(grid,),
        in_specs=[pl.BlockSpec((BLK, D), lambda i: (i, 0)), full((D, D))],
        out_specs=pl.BlockSpec((BLK, DA), lambda i: (i, 0)),
        out_shape=jax.ShapeDtypeStruct((n_pad, DA), jnp.float32),
    )(x_pad, W_v)

    sc_pass = _make_sc_pass(n_acc)
    acc_a = sc_pass(x_self_aug, row_p, col_p, zeros).reshape(NC, n_acc, DA)

    e_proj_aug = pl.pallas_call(
        _k2_body,
        grid=(grid,),
        in_specs=[pl.BlockSpec((NC, BLK, DA), lambda i: (0, i, 0)), full((D, D))],
        out_specs=pl.BlockSpec((BLK, DA), lambda i: (i, 0)),
        out_shape=jax.ShapeDtypeStruct((n_pad, DA), jnp.float32),
    )(acc_a, W_e)

    acc_b = sc_pass(e_proj_aug, col_p, row_p, zeros).reshape(NC, n_acc, DA)

    out = pl.pallas_call(
        _k3_body,
        grid=(grid,),
        in_specs=[
            pl.BlockSpec((BLK, DA), lambda i: (i, 0)),
            pl.BlockSpec((NC, BLK, DA), lambda i: (0, i, 0)),
            full((2 * D, D)),
            full((1, D)),
        ],
        out_specs=pl.BlockSpec((BLK, D), lambda i: (i, 0)),
        out_shape=jax.ShapeDtypeStruct((n_pad, D), jnp.float32),
    )(x_self_aug, acc_b, W_upd, b_upd.reshape(1, D))

    return out[:n]
